# baseline (speedup 1.0000x reference)
import jax
import jax.numpy as jnp
from jax import lax
from jax.experimental import pallas as pl
from jax.experimental.pallas import tpu as pltpu
from jax.experimental.pallas import tpu_sc as plsc

NU = 30000
NI = 20000
N = NU + NI
NP = 51200
D = 64
H = 32
E = 800000
NC = 2
NS = 16
GW = 128
PG = 392
SLABS = 14
SLABG = 28
EPAD = NS * PG * GW
RS = NP // NS
CH = RS // GW


def _body(embs, srcg, dstg, wg1, wg23, rowidx,
          outb, l1, l2,
          acc, src_v, dst_v, w_v, rowb, rowf,
          isem, gs0, gs1, gs2, gs3, ss0, ss1):
    gsems = (gs0, gs1, gs2, gs3)
    ssems = (ss0, ss1)
    c = lax.axis_index("c")
    s = lax.axis_index("s")
    r0 = s * RS

    def wait_g(b):
        pltpu.make_async_copy(embs.at[src_v.at[0]], rowb.at[b],
                              gsems[b]).wait()

    def wait_s(fb):
        pltpu.make_async_copy(rowf.at[fb], acc.at[dst_v.at[0]],
                              ssems[fb]).wait()

    def cvt_scale(b, fb, j):
        @pl.loop(0, GW // 16)
        def _eb(eb):
            wv = w_v[j, pl.ds(eb * 16, 16)]
            for k in range(16):
                e = eb * 16 + k
                w = wv[k]
                lo = rowb[b, e, pl.ds(0, 16)].astype(jnp.float32)
                hi = rowb[b, e, pl.ds(16, 16)].astype(jnp.float32)
                rowf[fb, e, pl.ds(0, 16)] = lo * w
                rowf[fb, e, pl.ds(16, 16)] = hi * w

    def edge_pass(table, weights):
        @pl.loop(0, SLABS)
        def _slab(t):
            m = s * SLABS + t
            i1 = pltpu.async_copy(srcg.at[c, m], src_v, isem)
            i2 = pltpu.async_copy(dstg.at[m], dst_v, isem)
            i3 = pltpu.async_copy(weights.at[m], w_v, isem)
            i1.wait()
            i2.wait()
            i3.wait()
            for b in range(3):
                pltpu.async_copy(table.at[src_v.at[b]], rowb.at[b], gsems[b])

            @pl.loop(0, SLABG // 4)
            def _quad(qd):
                for b in range(4):
                    j = qd * 4 + b
                    fb = b % 2
                    wait_g(b)
                    if b >= 2:
                        wait_s(fb)
                    else:
                        @pl.when(qd > 0)
                        def _ws():
                            wait_s(fb)
                    cvt_scale(b, fb, j)
                    pltpu.async_copy(rowf.at[fb], acc.at[dst_v.at[j]],
                                     ssems[fb], add=True)
                    bp = (b + 3) % 4
                    jr = j + 3
                    if b == 0:
                        pltpu.async_copy(table.at[src_v.at[jr]],
                                         rowb.at[bp], gsems[bp])
                    else:
                        @pl.when(qd < SLABG // 4 - 1)
                        def _rf(bp=bp, jr=jr):
                            pltpu.async_copy(table.at[src_v.at[jr]],
                                             rowb.at[bp], gsems[bp])

            for fb in range(2):
                wait_s(fb)

    def clear_acc():
        zv = jnp.zeros((16,), jnp.float32)

        @pl.loop(0, GW)
        def _zb(i):
            rowf[1, i, pl.ds(0, 16)] = zv
            rowf[1, i, pl.ds(16, 16)] = zv

        @pl.loop(0, CH)
        def _z(z):
            pltpu.sync_copy(rowf.at[1], acc.at[pl.ds(r0 + z * GW, GW)])

    def dump_acc(dst_hbm):
        pltpu.sync_copy(acc.at[pl.ds(r0, RS)],
                        dst_hbm.at[pl.ds(c * NP + r0, RS)])

    def dump_acc_bf(dst_hbm):
        def reclaim_out(b):
            pltpu.make_async_copy(rowb.at[b], dst_hbm.at[pl.ds(0, GW)],
                                  gsems[b]).wait()

        def chunk(k, b, in_loop):
            if in_loop:
                @pl.when(k >= 2)
                def _wo():
                    reclaim_out(b)
            else:
                reclaim_out(b)
            pltpu.sync_copy(acc.at[pl.ds(r0 + k * GW, GW)], rowf.at[b])

            @pl.loop(0, GW)
            def _cv(i):
                rowb[b, i, pl.ds(0, 16)] = \
                    rowf[b, i, pl.ds(0, 16)].astype(jnp.bfloat16)
                rowb[b, i, pl.ds(16, 16)] = \
                    rowf[b, i, pl.ds(16, 16)].astype(jnp.bfloat16)

            pltpu.async_copy(
                rowb.at[b],
                dst_hbm.at[pl.ds(c * NP + r0 + k * GW, GW)], gsems[b])

        @pl.loop(0, (CH - 1) // 2)
        def _pair(p):
            for b in range(2):
                chunk(p * 2 + b, b, True)

        chunk(CH - 1, 0, False)
        reclaim_out(0)
        reclaim_out(1)

    clear_acc()
    plsc.subcore_barrier()
    edge_pass(embs, wg1)
    plsc.subcore_barrier()
    dump_acc_bf(l1)
    plsc.subcore_barrier()

    clear_acc()
    plsc.subcore_barrier()
    edge_pass(l1, wg23)
    plsc.subcore_barrier()
    dump_acc_bf(l2)
    plsc.subcore_barrier()

    clear_acc()
    plsc.subcore_barrier()
    edge_pass(l2, wg23)

    pltpu.sync_copy(rowidx.at[s], src_v.at[pl.ds(0, CH)])

    def final_add(buf, scale):
        base = c * NP + r0
        for b in range(2):
            pltpu.async_copy(buf.at[pl.ds(base + b * GW, GW)],
                             rowb.at[b], gsems[b])

        def chunk(k, b, in_loop):
            wait_g(b)
            if in_loop:
                @pl.when(k >= 2)
                def _ws():
                    wait_s(b)
            else:
                wait_s(b)

            @pl.loop(0, GW)
            def _cv(i):
                lo = rowb[b, i, pl.ds(0, 16)].astype(jnp.float32)
                hi = rowb[b, i, pl.ds(16, 16)].astype(jnp.float32)
                rowf[b, i, pl.ds(0, 16)] = lo * scale
                rowf[b, i, pl.ds(16, 16)] = hi * scale

            pltpu.async_copy(rowf.at[b], acc.at[src_v.at[k]],
                             ssems[b], add=True)

            @pl.when(k < CH - 2)
            def _pf():
                pltpu.async_copy(buf.at[pl.ds(base + (k + 2) * GW, GW)],
                                 rowb.at[b], gsems[b])

        @pl.loop(0, (CH - 1) // 2)
        def _pair(p):
            for b in range(2):
                chunk(p * 2 + b, b, True)

        chunk(CH - 1, 0, False)
        for b in range(2):
            wait_s(b)

    final_add(embs, 0.25)
    final_add(l1, 1.0)
    final_add(l2, 1.0)
    plsc.subcore_barrier()
    dump_acc(outb)


@jax.jit
def _run(embs, srcg, dstg, wg1, wg23, rowidx):
    mesh = plsc.VectorSubcoreMesh(core_axis_name="c", subcore_axis_name="s",
                                  num_cores=NC, num_subcores=NS)
    f32 = jnp.float32
    bf16 = jnp.bfloat16
    out_type = (jax.ShapeDtypeStruct((NC * NP, H), f32),
                jax.ShapeDtypeStruct((NC * NP, H), bf16),
                jax.ShapeDtypeStruct((NC * NP, H), bf16))
    scratch = [
        pltpu.VMEM_SHARED((NP, H), f32),
        pltpu.VMEM((SLABG, GW), jnp.int32),
        pltpu.VMEM((SLABG, GW), jnp.int32),
        pltpu.VMEM((SLABG, GW), f32),
        pltpu.VMEM((4, GW, H), bf16),
        pltpu.VMEM((2, GW, H), f32),
        pltpu.SemaphoreType.DMA,
        pltpu.SemaphoreType.DMA,
        pltpu.SemaphoreType.DMA,
        pltpu.SemaphoreType.DMA,
        pltpu.SemaphoreType.DMA,
        pltpu.SemaphoreType.DMA,
        pltpu.SemaphoreType.DMA,
    ]
    return pl.kernel(
        _body, out_type=out_type, mesh=mesh, scratch_types=scratch,
        compiler_params=pltpu.CompilerParams(use_tc_tiling_on_sc=False),
    )(embs, srcg, dstg, wg1, wg23, rowidx)


def kernel(user_emb, item_emb, edge_index, edge_weight):
    all_emb = jnp.concatenate([user_emb, item_emb], axis=0)
    allp = jnp.concatenate(
        [all_emb, jnp.zeros((NP - N, D), jnp.float32)], axis=0)
    embs = (allp.astype(jnp.bfloat16)
            .reshape(NP, NC, H).transpose(1, 0, 2).reshape(NC * NP, H))

    pad = EPAD - E
    src = jnp.concatenate([edge_index[0], jnp.zeros((pad,), jnp.int32)])
    dst = jnp.concatenate([edge_index[1], jnp.zeros((pad,), jnp.int32)])
    w = jnp.concatenate([edge_weight, jnp.zeros((pad,), jnp.float32)])
    srcg = jnp.stack([src, src + NP]).reshape(NC, NS * SLABS, SLABG, GW)
    dstg = dst.reshape(NS * SLABS, SLABG, GW)
    wg1 = (w * 0.25).reshape(NS * SLABS, SLABG, GW)
    wg23 = w.reshape(NS * SLABS, SLABG, GW)
    rowidx = jnp.arange(NP, dtype=jnp.int32).reshape(NS, CH, GW)

    outb, _, _ = _run(embs, srcg, dstg, wg1, wg23, rowidx)
    final = outb.reshape(NC, NP, H).transpose(1, 0, 2).reshape(NP, D)
    return final[:NU], final[NU:N]


# speedup vs baseline: 1.6245x; 1.6245x over previous
import jax
import jax.numpy as jnp
from jax import lax
from jax.experimental import pallas as pl
from jax.experimental.pallas import tpu as pltpu
from jax.experimental.pallas import tpu_sc as plsc

NU = 30000
NI = 20000
N = NU + NI
NP = 51200
D = 64
H = 32
E = 800000
NC = 2
NS = 16
GW = 128
PG = 392
SLABS = 14
SLABG = 28
EPAD = NS * PG * GW
RS = NP // NS
CH = RS // GW


def _body(embs, srcg, dstg, wg1, wg23, rowidx,
          outb, l1, l2,
          acc, src_v, dst_v, w_v, rows, src_pre,
          isem, psem, gs0, gs1, gs2, gs3, ss0, ss1, ss2, ss3):
    gsems = (gs0, gs1, gs2, gs3)
    ssems = (ss0, ss1, ss2, ss3)
    c = lax.axis_index("c")
    s = lax.axis_index("s")
    r0 = s * RS

    def wait_g(b):
        pltpu.make_async_copy(embs.at[src_v.at[0]], rows.at[b],
                              gsems[b]).wait()

    def wait_s(b):
        pltpu.make_async_copy(rows.at[b], acc.at[dst_v.at[0]],
                              ssems[b]).wait()

    def scale_rows(b, j):
        @pl.loop(0, GW // 16)
        def _eb(eb):
            wv = w_v[j, pl.ds(eb * 16, 16)]
            for k in range(16):
                e = eb * 16 + k
                w = wv[k]
                rows[b, e, pl.ds(0, 16)] = rows[b, e, pl.ds(0, 16)] * w
                rows[b, e, pl.ds(16, 16)] = rows[b, e, pl.ds(16, 16)] * w

    def edge_pass(table, weights):
        @pl.loop(0, SLABS)
        def _slab(t):
            m = s * SLABS + t

            @pl.when(t == 0)
            def _first():
                pltpu.async_copy(srcg.at[c, m], src_v, isem)
                pltpu.async_copy(dstg.at[m], dst_v, isem)
                pltpu.async_copy(weights.at[m], w_v, isem)

            pltpu.make_async_copy(srcg.at[c, m], src_v, isem).wait()
            pltpu.make_async_copy(dstg.at[m], dst_v, isem).wait()
            pltpu.make_async_copy(weights.at[m], w_v, isem).wait()

            @pl.when(t == 0)
            def _prime():
                for b in range(3):
                    pltpu.async_copy(table.at[src_v.at[b]], rows.at[b],
                                     gsems[b])

            @pl.loop(0, SLABG // 4)
            def _quad(qd):
                for b in range(4):
                    j = qd * 4 + b
                    wait_g(b)
                    scale_rows(b, j)
                    pltpu.async_copy(rows.at[b], acc.at[dst_v.at[j]],
                                     ssems[b], add=True)
                    bp = (b + 3) % 4
                    jr = j + 3
                    if b == 0:
                        @pl.when(qd > 0)
                        def _w0():
                            wait_s(bp)
                        pltpu.async_copy(table.at[src_v.at[jr]],
                                         rows.at[bp], gsems[bp])
                    else:
                        @pl.when(qd < SLABG // 4 - 1)
                        def _rf(bp=bp, jr=jr):
                            wait_s(bp)
                            pltpu.async_copy(table.at[src_v.at[jr]],
                                             rows.at[bp], gsems[bp])

            @pl.when(t < SLABS - 1)
            def _tail():
                pltpu.async_copy(srcg.at[c, m + 1, pl.ds(0, 4)], src_pre,
                                 psem)
                pltpu.make_async_copy(srcg.at[c, m + 1, pl.ds(0, 4)],
                                      src_pre, psem).wait()
                for bi in range(3):
                    wait_s(bi)
                    pltpu.async_copy(table.at[src_pre.at[bi]], rows.at[bi],
                                     gsems[bi])
                wait_s(3)
                pltpu.async_copy(srcg.at[c, m + 1], src_v, isem)
                pltpu.async_copy(dstg.at[m + 1], dst_v, isem)
                pltpu.async_copy(weights.at[m + 1], w_v, isem)

            @pl.when(t == SLABS - 1)
            def _drain():
                for b in range(4):
                    wait_s(b)

    def clear_acc():
        zv = jnp.zeros((16,), jnp.float32)

        @pl.loop(0, GW)
        def _zb(i):
            rows[3, i, pl.ds(0, 16)] = zv
            rows[3, i, pl.ds(16, 16)] = zv

        @pl.loop(0, CH)
        def _z(z):
            pltpu.sync_copy(rows.at[3], acc.at[pl.ds(r0 + z * GW, GW)])

    def dump_acc(dst_hbm):
        pltpu.sync_copy(acc.at[pl.ds(r0, RS)],
                        dst_hbm.at[pl.ds(c * NP + r0, RS)])

    clear_acc()
    plsc.subcore_barrier()
    edge_pass(embs, wg1)
    plsc.subcore_barrier()
    dump_acc(l1)

    clear_acc()
    plsc.subcore_barrier()
    edge_pass(l1, wg23)
    plsc.subcore_barrier()
    dump_acc(l2)

    clear_acc()
    plsc.subcore_barrier()
    edge_pass(l2, wg23)

    pltpu.sync_copy(rowidx.at[s], src_v.at[pl.ds(0, CH)])

    def final_add(buf, scale):
        base = c * NP + r0
        for b in range(2):
            pltpu.async_copy(buf.at[pl.ds(base + b * GW, GW)],
                             rows.at[b], gsems[b])

        def chunk(k, b):
            wait_g(b)
            if scale:
                @pl.loop(0, GW)
                def _sc(i):
                    rows[b, i, pl.ds(0, 16)] = rows[b, i, pl.ds(0, 16)] * 0.25
                    rows[b, i, pl.ds(16, 16)] = rows[b, i, pl.ds(16, 16)] * 0.25
            pltpu.sync_copy(rows.at[b], acc.at[src_v.at[k]], add=True)

            @pl.when(k < CH - 2)
            def _pf():
                pltpu.async_copy(buf.at[pl.ds(base + (k + 2) * GW, GW)],
                                 rows.at[b], gsems[b])

        @pl.loop(0, (CH - 1) // 2)
        def _pair(p):
            for b in range(2):
                chunk(p * 2 + b, b)

        chunk(CH - 1, 0)

    final_add(embs, True)
    final_add(l1, False)
    final_add(l2, False)
    plsc.subcore_barrier()
    dump_acc(outb)


@jax.jit
def _run(embs, srcg, dstg, wg1, wg23, rowidx):
    mesh = plsc.VectorSubcoreMesh(core_axis_name="c", subcore_axis_name="s",
                                  num_cores=NC, num_subcores=NS)
    f32 = jnp.float32
    out_type = (jax.ShapeDtypeStruct((NC * NP, H), f32),
                jax.ShapeDtypeStruct((NC * NP, H), f32),
                jax.ShapeDtypeStruct((NC * NP, H), f32))
    scratch = [
        pltpu.VMEM_SHARED((NP, H), f32),
        pltpu.VMEM((SLABG, GW), jnp.int32),
        pltpu.VMEM((SLABG, GW), jnp.int32),
        pltpu.VMEM((SLABG, GW), f32),
        pltpu.VMEM((4, GW, H), f32),
        pltpu.VMEM((4, GW), jnp.int32),
        pltpu.SemaphoreType.DMA,
        pltpu.SemaphoreType.DMA,
        pltpu.SemaphoreType.DMA,
        pltpu.SemaphoreType.DMA,
        pltpu.SemaphoreType.DMA,
        pltpu.SemaphoreType.DMA,
        pltpu.SemaphoreType.DMA,
        pltpu.SemaphoreType.DMA,
        pltpu.SemaphoreType.DMA,
        pltpu.SemaphoreType.DMA,
    ]
    return pl.kernel(
        _body, out_type=out_type, mesh=mesh, scratch_types=scratch,
        compiler_params=pltpu.CompilerParams(use_tc_tiling_on_sc=False),
    )(embs, srcg, dstg, wg1, wg23, rowidx)


def kernel(user_emb, item_emb, edge_index, edge_weight):
    all_emb = jnp.concatenate([user_emb, item_emb], axis=0)
    allp = jnp.concatenate(
        [all_emb, jnp.zeros((NP - N, D), jnp.float32)], axis=0)
    embs = allp.reshape(NP, NC, H).transpose(1, 0, 2).reshape(NC * NP, H)

    pad = EPAD - E
    src = jnp.concatenate([edge_index[0], jnp.zeros((pad,), jnp.int32)])
    dst = jnp.concatenate([edge_index[1], jnp.zeros((pad,), jnp.int32)])
    w = jnp.concatenate([edge_weight, jnp.zeros((pad,), jnp.float32)])
    srcg = jnp.stack([src, src + NP]).reshape(NC, NS * SLABS, SLABG, GW)
    dstg = dst.reshape(NS * SLABS, SLABG, GW)
    wg1 = (w * 0.25).reshape(NS * SLABS, SLABG, GW)
    wg23 = w.reshape(NS * SLABS, SLABG, GW)
    rowidx = jnp.arange(NP, dtype=jnp.int32).reshape(NS, CH, GW)

    outb, _, _ = _run(embs, srcg, dstg, wg1, wg23, rowidx)
    final = outb.reshape(NC, NP, H).transpose(1, 0, 2).reshape(NP, D)
    return final[:NU], final[NU:N]
